# baseline (speedup 1.0000x reference)
import functools

import jax
import jax.numpy as jnp
from jax import lax
from jax.experimental import pallas as pl
from jax.experimental.pallas import tpu as pltpu
from jax.experimental.pallas import tpu_sc as plsc

_NC = 2
_NS = 16
_NW = _NC * _NS
_L = 16
_CHUNK = 128
_VCHUNK = 128


def _relayout_body(V, D, n_chunks, per_w, tbl_a, tbl_b, tail_a, tail_b,
                   out_a, out_b, blk, tv, sem):
    wid = lax.axis_index("s") * _NC + lax.axis_index("c")
    lanes = lax.iota(jnp.int32, _L)
    tail_base = n_chunks * _VCHUNK * D

    for tbl, tail, out in ((tbl_a, tail_a, out_a), (tbl_b, tail_b, out_b)):
        lo = wid * per_w
        hi = jnp.minimum(lo + per_w, n_chunks)

        def chunk(c, _, tbl=tbl, out=out):
            start = pl.multiple_of(c * _VCHUNK, _VCHUNK)
            pltpu.sync_copy(tbl.at[:, pl.ds(start, _VCHUNK)], blk)

            def col(j, _2):
                vals = plsc.load_gather(
                    blk, [lanes, jnp.full((_L,), 0, jnp.int32) + j])
                tv[pl.ds(j * D, D)] = vals
                return _2

            lax.fori_loop(0, _VCHUNK, col, None)
            pltpu.sync_copy(tv, out.at[pl.ds(c * _VCHUNK * D, _VCHUNK * D)])
            return _

        lax.fori_loop(lo, hi, chunk, None)

        @pl.when(wid == 0)
        def _copy_tail(tail=tail, out=out):
            pltpu.sync_copy(tail, out.at[pl.ds(tail_base, tail.shape[0])])


def _gather_body(n_per_w, D, idx0_hbm, idx1_hbm, idx2_hbm, emb_in_hbm,
                 emb_w_hbm, emb_b_hbm, out_hbm,
                 idx0_v, idx1_v, idx2_v, e0_v, e1_v, w_v, b_v, out_v, sem):
    wid = lax.axis_index("s") * _NC + lax.axis_index("c")
    base = wid * n_per_w

    pltpu.sync_copy(idx0_hbm.at[pl.ds(base, n_per_w)], idx0_v)
    pltpu.sync_copy(idx1_hbm.at[pl.ds(base, n_per_w)], idx1_v)
    pltpu.sync_copy(idx2_hbm.at[pl.ds(base, n_per_w)], idx2_v)

    copies = []
    for k in range(0, n_per_w, _CHUNK):
        sl = pl.ds(k, _CHUNK)
        copies.append(pltpu.async_copy(
            emb_in_hbm.at[idx0_v.at[sl]], e0_v.at[sl], sem))
        copies.append(pltpu.async_copy(
            emb_in_hbm.at[idx1_v.at[sl]], e1_v.at[sl], sem))
        copies.append(pltpu.async_copy(
            emb_w_hbm.at[idx2_v.at[sl]], w_v.at[sl], sem))
        copies.append(pltpu.async_copy(
            emb_b_hbm.at[idx2_v.at[sl]], b_v.at[sl], sem))
    for c in copies:
        c.wait()

    lanes = lax.iota(jnp.int32, _L)

    def group(g, _):
        rows = g * _L + lanes
        acc = jnp.zeros((_L,), jnp.float32)
        for d in range(D):
            col = jnp.full((_L,), d, jnp.int32)
            a0 = plsc.load_gather(e0_v, [rows, col])
            a1 = plsc.load_gather(e1_v, [rows, col])
            aw = plsc.load_gather(w_v, [rows, col])
            acc = acc + (a0 + a1) * aw
        logit = acc * 0.5 + b_v[pl.ds(g * _L, _L)]
        out_v[pl.ds(g * _L, _L)] = 1.0 / (1.0 + jnp.exp(-logit))
        return _

    lax.fori_loop(0, n_per_w // _L, group, None)

    pltpu.sync_copy(out_v, out_hbm.at[pl.ds(base, n_per_w)])


def kernel(x, emb_in, emb_out_w, emb_out_b):
    B = x.shape[0]
    V, D = emb_in.shape
    n_chunks = V // _VCHUNK
    tail_n = V - n_chunks * _VCHUNK
    VP = (n_chunks + (1 if tail_n else 0)) * _VCHUNK
    per_w = -(-n_chunks // _NW)
    n_per_w = B // _NW

    mesh = plsc.VectorSubcoreMesh(core_axis_name="c", subcore_axis_name="s")

    relayout = pl.kernel(
        functools.partial(_relayout_body, V, D, n_chunks, per_w),
        out_type=(jax.ShapeDtypeStruct((VP * D,), jnp.float32),
                  jax.ShapeDtypeStruct((VP * D,), jnp.float32)),
        mesh=mesh,
        scratch_types=[
            pltpu.VMEM((D, _VCHUNK), jnp.float32),
            pltpu.VMEM((_VCHUNK * D,), jnp.float32),
            pltpu.SemaphoreType.DMA,
        ],
        compiler_params=pltpu.CompilerParams(
            needs_layout_passes=False, use_tc_tiling_on_sc=True),
    )
    tail_in = emb_in[n_chunks * _VCHUNK:].reshape(-1)
    tail_w = emb_out_w[n_chunks * _VCHUNK:].reshape(-1)
    lin_in_flat, lin_w_flat = relayout(emb_in.T, emb_out_w.T, tail_in, tail_w)
    lin_in = lin_in_flat.reshape(VP, D)
    lin_w = lin_w_flat.reshape(VP, D)

    gather = pl.kernel(
        functools.partial(_gather_body, n_per_w, D),
        out_type=jax.ShapeDtypeStruct((B,), jnp.float32),
        mesh=mesh,
        scratch_types=[
            pltpu.VMEM((n_per_w,), jnp.int32),
            pltpu.VMEM((n_per_w,), jnp.int32),
            pltpu.VMEM((n_per_w,), jnp.int32),
            pltpu.VMEM((n_per_w, D), jnp.float32),
            pltpu.VMEM((n_per_w, D), jnp.float32),
            pltpu.VMEM((n_per_w, D), jnp.float32),
            pltpu.VMEM((n_per_w,), jnp.float32),
            pltpu.VMEM((n_per_w,), jnp.float32),
            pltpu.SemaphoreType.DMA,
        ],
        compiler_params=pltpu.CompilerParams(
            needs_layout_passes=False, use_tc_tiling_on_sc=False),
    )
    out = gather(x[:, 0], x[:, 1], x[:, 2], lin_in, lin_w,
                 emb_out_b.reshape(V))
    return out.reshape(B, 1)
